# baseline (speedup 1.0000x reference)
import functools

import jax
import jax.numpy as jnp
from jax import lax
from jax.experimental import pallas as pl
from jax.experimental.pallas import tpu as pltpu
from jax.experimental.pallas import tpu_sc as plsc

VOCAB = 100000
EMBED = 128
HID = 128
B = 4096
T = 32
NCLS = 10
NCLS_PAD = 128

BT = B * T
CH = 128

BB = 512
NSB = 1
SBB = BB // NSB


def _make_sc_gather(n_rows):
    info = plsc.get_sparse_core_info()
    nw = info.num_cores * info.num_subcores
    per_w = n_rows // nw
    n_ch = per_w // CH

    mesh = plsc.VectorSubcoreMesh(core_axis_name="c", subcore_axis_name="s")

    @functools.partial(
        pl.kernel,
        mesh=mesh,
        out_type=jax.ShapeDtypeStruct((n_rows, EMBED), jnp.float32),
        scratch_types=[
            pltpu.VMEM((per_w,), jnp.int32),
            pltpu.VMEM((CH, EMBED), jnp.float32),
            pltpu.VMEM((CH, EMBED), jnp.float32),
            pltpu.SemaphoreType.DMA,
            pltpu.SemaphoreType.DMA,
        ],
    )
    def gather_k(idx_hbm, table_hbm, out_hbm, idx_v, rows0, rows1, sem0, sem1):
        wid = lax.axis_index("s") * info.num_cores + lax.axis_index("c")
        base = wid * per_w
        pltpu.sync_copy(idx_hbm.at[pl.ds(base, per_w)], idx_v)

        rows = (rows0, rows1)
        sems = (sem0, sem1)

        def issue(j, slot):
            off = pl.multiple_of(j * CH, CH)
            pltpu.async_copy(
                table_hbm.at[idx_v.at[pl.ds(off, CH)]], rows[slot], sems[slot]
            )

        def drain(j, slot):
            off = pl.multiple_of(j * CH, CH)
            pltpu.make_async_copy(
                table_hbm.at[idx_v.at[pl.ds(off, CH)]], rows[slot], sems[slot]
            ).wait()
            pltpu.sync_copy(rows[slot], out_hbm.at[pl.ds(base + off, CH)])

        issue(0, 0)

        def body(k, _):
            j0 = k * 2
            for slot in range(2):
                j = j0 + slot

                @pl.when(j + 1 < n_ch)
                def _():
                    issue(j + 1, (slot + 1) % 2)

                drain(j, slot)
            return 0

        lax.fori_loop(0, n_ch // 2, body, 0)

    return gather_k


_sc_gather_cache = {}


def _get_sc_gather(n_rows):
    if n_rows not in _sc_gather_cache:
        _sc_gather_cache[n_rows] = _make_sc_gather(n_rows)
    return _sc_gather_cache[n_rows]


def _lstm_body(emb_ref, w0_ref, w1_ref,
               fcw_ref, fcb_ref, out_ref, h2flat_ref):
    w0 = w0_ref[...]
    w1 = w1_ref[...]
    ones8 = jnp.ones((BB, 8), jnp.bfloat16)

    def act(g, c):
        ti = jnp.tanh(g[:, :HID])
        tf = jnp.tanh(g[:, HID:2 * HID])
        tg = jnp.tanh(g[:, 2 * HID:3 * HID])
        to = jnp.tanh(g[:, 3 * HID:])
        c_new = 0.5 * ((c + tf * c) + (tg + ti * tg))
        tc = jnp.tanh(c_new)
        h_new = 0.5 * (tc + to * tc)
        return h_new, c_new

    def step(t, carry):
        h1, c1, h2, c2 = carry
        x_t = emb_ref[:, pl.ds(t * EMBED, EMBED)].astype(jnp.bfloat16)
        g1 = jnp.dot(jnp.concatenate([x_t, h1, ones8], axis=1), w0,
                     preferred_element_type=jnp.float32)
        h1f, c1 = act(g1, c1)
        h1 = h1f.astype(jnp.bfloat16)
        g2 = jnp.dot(jnp.concatenate([h1, h2, ones8], axis=1), w1,
                     preferred_element_type=jnp.float32)
        h2f, c2 = act(g2, c2)
        h2 = h2f.astype(jnp.bfloat16)
        h2flat_ref[:, pl.ds(t * HID, HID)] = h2
        return h1, c1, h2, c2

    zb = jnp.zeros((BB, HID), jnp.bfloat16)
    zf = jnp.zeros((BB, HID), jnp.float32)
    lax.fori_loop(0, T, step, (zb, zf, zb, zf), unroll=16)

    logits = jnp.dot(h2flat_ref[...], fcw_ref[...],
                     preferred_element_type=jnp.float32) + fcb_ref[...]
    m = jnp.max(logits, axis=1, keepdims=True)
    e = jnp.exp(logits - m)
    out_ref[...] = e / jnp.sum(e, axis=1, keepdims=True)


def _tc_lstm(emb2d, w0, w1, fcw, fcb):
    n_b = emb2d.shape[0]
    grid = (n_b // BB,)
    return pl.pallas_call(
        _lstm_body,
        grid=grid,
        in_specs=[
            pl.BlockSpec((BB, T * EMBED), lambda i: (i, 0)),
            pl.BlockSpec((2 * HID + 8, 4 * HID), lambda i: (0, 0)),
            pl.BlockSpec((2 * HID + 8, 4 * HID), lambda i: (0, 0)),
            pl.BlockSpec((T * HID, NCLS_PAD), lambda i: (0, 0)),
            pl.BlockSpec((1, NCLS_PAD), lambda i: (0, 0)),
        ],
        out_specs=pl.BlockSpec((BB, NCLS_PAD), lambda i: (i, 0)),
        out_shape=jax.ShapeDtypeStruct((n_b, NCLS_PAD), jnp.float32),
        scratch_shapes=[pltpu.VMEM((BB, T * HID), jnp.bfloat16)],
    )(emb2d, w0, w1, fcw, fcb)


def kernel(x, emb_table, Wih0, Whh0, bih0, bhh0, Wih1, Whh1, bih1, bhh1,
           fcW, fcb):
    gate_scale = jnp.concatenate([
        jnp.full((2 * HID,), 0.5, jnp.float32),
        jnp.ones((HID,), jnp.float32),
        jnp.full((HID,), 0.5, jnp.float32),
    ]).reshape(1, 4 * HID)
    b0 = ((bih0 + bhh0).reshape(1, 4 * HID)) * gate_scale / 8.0
    b1 = ((bih1 + bhh1).reshape(1, 4 * HID)) * gate_scale / 8.0
    w0 = jnp.concatenate(
        [jnp.concatenate([Wih0.T, Whh0.T], axis=0) * gate_scale,
         jnp.broadcast_to(b0, (8, 4 * HID))], axis=0).astype(jnp.bfloat16)
    w1 = jnp.concatenate(
        [jnp.concatenate([Wih1.T, Whh1.T], axis=0) * gate_scale,
         jnp.broadcast_to(b1, (8, 4 * HID))], axis=0).astype(jnp.bfloat16)

    fcw = (jnp.zeros((T * HID, NCLS_PAD), jnp.float32)
           .at[:, :NCLS].set(fcW.T).astype(jnp.bfloat16))
    fcb_pad = jnp.full((1, NCLS_PAD), -1e30, jnp.float32).at[0, :NCLS].set(fcb)

    nsplit = 4
    bh = B // nsplit
    gath = _get_sc_gather(bh * T)
    halves = [gath(x[k * bh:(k + 1) * bh].reshape(-1).astype(jnp.int32),
                   emb_table) for k in range(nsplit)]
    probs = [_tc_lstm(g.reshape(bh, T * EMBED), w0, w1,
                      fcw, fcb_pad) for g in halves]
    return jnp.concatenate(probs, axis=0)[:, :NCLS]


# speedup vs baseline: 1.2492x; 1.2492x over previous
import functools

import jax
import jax.numpy as jnp
from jax import lax
from jax.experimental import pallas as pl
from jax.experimental.pallas import tpu as pltpu
from jax.experimental.pallas import tpu_sc as plsc

VOCAB = 100000
EMBED = 128
HID = 128
B = 4096
T = 32
NCLS = 10
NCLS_PAD = 128

BT = B * T
CH = 128

BB = 512
NSB = 1
SBB = BB // NSB


def _make_sc_gather(n_rows):
    info = plsc.get_sparse_core_info()
    nw = info.num_cores * info.num_subcores
    per_w = n_rows // nw
    n_ch = per_w // CH

    mesh = plsc.VectorSubcoreMesh(core_axis_name="c", subcore_axis_name="s")

    @functools.partial(
        pl.kernel,
        mesh=mesh,
        out_type=jax.ShapeDtypeStruct((n_rows, EMBED), jnp.float32),
        scratch_types=[
            pltpu.VMEM((per_w,), jnp.int32),
            pltpu.VMEM((CH, EMBED), jnp.float32),
            pltpu.VMEM((CH, EMBED), jnp.float32),
            pltpu.SemaphoreType.DMA,
            pltpu.SemaphoreType.DMA,
        ],
    )
    def gather_k(idx_hbm, table_hbm, out_hbm, idx_v, rows0, rows1, sem0, sem1):
        wid = lax.axis_index("s") * info.num_cores + lax.axis_index("c")
        base = wid * per_w
        pltpu.sync_copy(idx_hbm.at[pl.ds(base, per_w)], idx_v)

        rows = (rows0, rows1)
        sems = (sem0, sem1)

        def issue(j, slot):
            off = pl.multiple_of(j * CH, CH)
            pltpu.async_copy(
                table_hbm.at[idx_v.at[pl.ds(off, CH)]], rows[slot], sems[slot]
            )

        def drain(j, slot):
            off = pl.multiple_of(j * CH, CH)
            pltpu.make_async_copy(
                table_hbm.at[idx_v.at[pl.ds(off, CH)]], rows[slot], sems[slot]
            ).wait()
            pltpu.sync_copy(rows[slot], out_hbm.at[pl.ds(base + off, CH)])

        issue(0, 0)

        def body(k, _):
            j0 = k * 2
            for slot in range(2):
                j = j0 + slot

                @pl.when(j + 1 < n_ch)
                def _():
                    issue(j + 1, (slot + 1) % 2)

                drain(j, slot)
            return 0

        lax.fori_loop(0, n_ch // 2, body, 0)

    return gather_k


_sc_gather_cache = {}


def _get_sc_gather(n_rows):
    if n_rows not in _sc_gather_cache:
        _sc_gather_cache[n_rows] = _make_sc_gather(n_rows)
    return _sc_gather_cache[n_rows]


def _lstm_body(emb_ref, w0_ref, b0_ref, w1_ref, b1_ref,
               fcw_ref, fcb_ref, out_ref, h2flat_ref):
    w0 = w0_ref[...]
    b0 = b0_ref[...]
    w1 = w1_ref[...]
    b1 = b1_ref[...]

    def act(g, c):
        ti = jnp.tanh(g[:, :HID])
        tf = jnp.tanh(g[:, HID:2 * HID])
        tg = jnp.tanh(g[:, 2 * HID:3 * HID])
        to = jnp.tanh(g[:, 3 * HID:])
        c_new = 0.5 * ((c + tf * c) + (tg + ti * tg))
        tc = jnp.tanh(c_new)
        h_new = 0.5 * (tc + to * tc)
        return h_new, c_new

    def step(t, carry):
        h1, c1, h2, c2 = carry
        x_t = emb_ref[:, pl.ds(t * EMBED, EMBED)].astype(jnp.bfloat16)
        g1 = jnp.dot(jnp.concatenate([x_t, h1], axis=1), w0,
                     preferred_element_type=jnp.float32) + b0
        h1f, c1 = act(g1, c1)
        h1 = h1f.astype(jnp.bfloat16)
        g2 = jnp.dot(jnp.concatenate([h1, h2], axis=1), w1,
                     preferred_element_type=jnp.float32) + b1
        h2f, c2 = act(g2, c2)
        h2 = h2f.astype(jnp.bfloat16)
        h2flat_ref[:, pl.ds(t * HID, HID)] = h2
        return h1, c1, h2, c2

    zb = jnp.zeros((BB, HID), jnp.bfloat16)
    zf = jnp.zeros((BB, HID), jnp.float32)
    lax.fori_loop(0, T, step, (zb, zf, zb, zf), unroll=16)

    logits = jnp.dot(h2flat_ref[...], fcw_ref[...],
                     preferred_element_type=jnp.float32) + fcb_ref[...]
    m = jnp.max(logits, axis=1, keepdims=True)
    e = jnp.exp(logits - m)
    out_ref[...] = e / jnp.sum(e, axis=1, keepdims=True)


def _tc_lstm(emb2d, w0, b0, w1, b1, fcw, fcb):
    n_b = emb2d.shape[0]
    grid = (n_b // BB,)
    return pl.pallas_call(
        _lstm_body,
        grid=grid,
        in_specs=[
            pl.BlockSpec((BB, T * EMBED), lambda i: (i, 0)),
            pl.BlockSpec((2 * HID, 4 * HID), lambda i: (0, 0)),
            pl.BlockSpec((1, 4 * HID), lambda i: (0, 0)),
            pl.BlockSpec((2 * HID, 4 * HID), lambda i: (0, 0)),
            pl.BlockSpec((1, 4 * HID), lambda i: (0, 0)),
            pl.BlockSpec((T * HID, NCLS_PAD), lambda i: (0, 0)),
            pl.BlockSpec((1, NCLS_PAD), lambda i: (0, 0)),
        ],
        out_specs=pl.BlockSpec((BB, NCLS_PAD), lambda i: (i, 0)),
        out_shape=jax.ShapeDtypeStruct((n_b, NCLS_PAD), jnp.float32),
        scratch_shapes=[pltpu.VMEM((BB, T * HID), jnp.bfloat16)],
    )(emb2d, w0, b0, w1, b1, fcw, fcb)


def kernel(x, emb_table, Wih0, Whh0, bih0, bhh0, Wih1, Whh1, bih1, bhh1,
           fcW, fcb):
    gate_scale = jnp.concatenate([
        jnp.full((2 * HID,), 0.5, jnp.float32),
        jnp.ones((HID,), jnp.float32),
        jnp.full((HID,), 0.5, jnp.float32),
    ]).reshape(1, 4 * HID)
    w0 = (jnp.concatenate([Wih0.T, Whh0.T], axis=0)
          * gate_scale).astype(jnp.bfloat16)
    b0 = ((bih0 + bhh0).reshape(1, 4 * HID)) * gate_scale
    w1 = (jnp.concatenate([Wih1.T, Whh1.T], axis=0)
          * gate_scale).astype(jnp.bfloat16)
    b1 = ((bih1 + bhh1).reshape(1, 4 * HID)) * gate_scale

    fcw = (jnp.zeros((T * HID, NCLS_PAD), jnp.float32)
           .at[:, :NCLS].set(fcW.T).astype(jnp.bfloat16))
    fcb_pad = jnp.full((1, NCLS_PAD), -1e30, jnp.float32).at[0, :NCLS].set(fcb)

    nsplit = 4
    bh = B // nsplit
    gath = _get_sc_gather(bh * T)
    halves = [gath(x[k * bh:(k + 1) * bh].reshape(-1).astype(jnp.int32),
                   emb_table) for k in range(nsplit)]
    probs = [_tc_lstm(g.reshape(bh, T * EMBED), w0, b0, w1, b1,
                      fcw, fcb_pad) for g in halves]
    return jnp.concatenate(probs, axis=0)[:, :NCLS]


# speedup vs baseline: 1.8153x; 1.4531x over previous
import functools

import jax
import jax.numpy as jnp
from jax import lax
from jax.experimental import pallas as pl
from jax.experimental.pallas import tpu as pltpu
from jax.experimental.pallas import tpu_sc as plsc

VOCAB = 100000
EMBED = 128
HID = 128
B = 4096
T = 32
NCLS = 10
NCLS_PAD = 128

BT = B * T
CH = 128

BB = 512
NSB = 1
SBB = BB // NSB


def _make_sc_gather(n_rows):
    info = plsc.get_sparse_core_info()
    nw = info.num_cores * info.num_subcores
    per_w = n_rows // nw
    n_ch = per_w // CH

    mesh = plsc.VectorSubcoreMesh(core_axis_name="c", subcore_axis_name="s")

    @functools.partial(
        pl.kernel,
        mesh=mesh,
        out_type=jax.ShapeDtypeStruct((n_rows, EMBED), jnp.float32),
        scratch_types=[
            pltpu.VMEM((per_w,), jnp.int32),
            pltpu.VMEM((CH, EMBED), jnp.float32),
            pltpu.VMEM((CH, EMBED), jnp.float32),
            pltpu.SemaphoreType.DMA,
            pltpu.SemaphoreType.DMA,
        ],
    )
    def gather_k(idx_hbm, table_hbm, out_hbm, idx_v, rows0, rows1, sem0, sem1):
        wid = lax.axis_index("s") * info.num_cores + lax.axis_index("c")
        base = wid * per_w
        pltpu.sync_copy(idx_hbm.at[pl.ds(base, per_w)], idx_v)

        rows = (rows0, rows1)
        sems = (sem0, sem1)

        def issue(j, slot):
            off = pl.multiple_of(j * CH, CH)
            pltpu.async_copy(
                table_hbm.at[idx_v.at[pl.ds(off, CH)]], rows[slot], sems[slot]
            )

        def drain(j, slot):
            off = pl.multiple_of(j * CH, CH)
            pltpu.make_async_copy(
                table_hbm.at[idx_v.at[pl.ds(off, CH)]], rows[slot], sems[slot]
            ).wait()
            pltpu.sync_copy(rows[slot], out_hbm.at[pl.ds(base + off, CH)])

        issue(0, 0)

        def body(k, _):
            j0 = k * 2
            for slot in range(2):
                j = j0 + slot

                @pl.when(j + 1 < n_ch)
                def _():
                    issue(j + 1, (slot + 1) % 2)

                drain(j, slot)
            return 0

        lax.fori_loop(0, n_ch // 2, body, 0)

    return gather_k


_sc_gather_cache = {}


def _get_sc_gather(n_rows):
    if n_rows not in _sc_gather_cache:
        _sc_gather_cache[n_rows] = _make_sc_gather(n_rows)
    return _sc_gather_cache[n_rows]


def _lstm_body(emb_ref, w0_ref, b0_ref, w1_ref, b1_ref,
               fcw_ref, fcb_ref, out_ref, h2flat_ref):
    w0 = w0_ref[...]
    b0 = b0_ref[...]
    w1 = w1_ref[...]
    b1 = b1_ref[...]

    def act(g, c):
        ti = jnp.tanh(g[:, :HID])
        tf = jnp.tanh(g[:, HID:2 * HID])
        tg = jnp.tanh(g[:, 2 * HID:3 * HID])
        to = jnp.tanh(g[:, 3 * HID:])
        c_new = 0.5 * ((c + tf * c) + (tg + ti * tg))
        tc = jnp.tanh(c_new)
        h_new = 0.5 * (tc + to * tc)
        return h_new, c_new

    def step(t, carry):
        h1, c1, h2, c2 = carry
        x_t = jnp.squeeze(emb_ref[pl.ds(t, 1)], axis=0).astype(jnp.bfloat16)
        g1 = jnp.dot(jnp.concatenate([x_t, h1], axis=1), w0,
                     preferred_element_type=jnp.float32) + b0
        h1f, c1 = act(g1, c1)
        h1 = h1f.astype(jnp.bfloat16)
        g2 = jnp.dot(jnp.concatenate([h1, h2], axis=1), w1,
                     preferred_element_type=jnp.float32) + b1
        h2f, c2 = act(g2, c2)
        h2 = h2f.astype(jnp.bfloat16)
        h2flat_ref[:, pl.ds(t * HID, HID)] = h2
        return h1, c1, h2, c2

    zb = jnp.zeros((BB, HID), jnp.bfloat16)
    zf = jnp.zeros((BB, HID), jnp.float32)
    lax.fori_loop(0, T, step, (zb, zf, zb, zf), unroll=16)

    logits = jnp.dot(h2flat_ref[...], fcw_ref[...],
                     preferred_element_type=jnp.float32) + fcb_ref[...]
    m = jnp.max(logits, axis=1, keepdims=True)
    e = jnp.exp(logits - m)
    out_ref[...] = e / jnp.sum(e, axis=1, keepdims=True)


def _tc_lstm(emb3, w0, b0, w1, b1, fcw, fcb):
    n_b = emb3.shape[1]
    grid = (n_b // BB,)
    return pl.pallas_call(
        _lstm_body,
        grid=grid,
        in_specs=[
            pl.BlockSpec((T, BB, EMBED), lambda i: (0, i, 0)),
            pl.BlockSpec((2 * HID, 4 * HID), lambda i: (0, 0)),
            pl.BlockSpec((1, 4 * HID), lambda i: (0, 0)),
            pl.BlockSpec((2 * HID, 4 * HID), lambda i: (0, 0)),
            pl.BlockSpec((1, 4 * HID), lambda i: (0, 0)),
            pl.BlockSpec((T * HID, NCLS_PAD), lambda i: (0, 0)),
            pl.BlockSpec((1, NCLS_PAD), lambda i: (0, 0)),
        ],
        out_specs=pl.BlockSpec((BB, NCLS_PAD), lambda i: (i, 0)),
        out_shape=jax.ShapeDtypeStruct((n_b, NCLS_PAD), jnp.float32),
        scratch_shapes=[pltpu.VMEM((BB, T * HID), jnp.bfloat16)],
    )(emb3, w0, b0, w1, b1, fcw, fcb)


def kernel(x, emb_table, Wih0, Whh0, bih0, bhh0, Wih1, Whh1, bih1, bhh1,
           fcW, fcb):
    gate_scale = jnp.concatenate([
        jnp.full((2 * HID,), 0.5, jnp.float32),
        jnp.ones((HID,), jnp.float32),
        jnp.full((HID,), 0.5, jnp.float32),
    ]).reshape(1, 4 * HID)
    w0 = (jnp.concatenate([Wih0.T, Whh0.T], axis=0)
          * gate_scale).astype(jnp.bfloat16)
    b0 = ((bih0 + bhh0).reshape(1, 4 * HID)) * gate_scale
    w1 = (jnp.concatenate([Wih1.T, Whh1.T], axis=0)
          * gate_scale).astype(jnp.bfloat16)
    b1 = ((bih1 + bhh1).reshape(1, 4 * HID)) * gate_scale

    fcw = (jnp.zeros((T * HID, NCLS_PAD), jnp.float32)
           .at[:, :NCLS].set(fcW.T).astype(jnp.bfloat16))
    fcb_pad = jnp.full((1, NCLS_PAD), -1e30, jnp.float32).at[0, :NCLS].set(fcb)

    nsplit = 4
    bh = B // nsplit
    gath = _get_sc_gather(bh * T)
    halves = [gath(x[k * bh:(k + 1) * bh].astype(jnp.int32).T.reshape(-1),
                   emb_table) for k in range(nsplit)]
    probs = [_tc_lstm(g.reshape(T, bh, EMBED), w0, b0, w1, b1,
                      fcw, fcb_pad) for g in halves]
    return jnp.concatenate(probs, axis=0)[:, :NCLS]
